# baseline (speedup 1.0000x reference)
import functools

import jax
import jax.numpy as jnp
from jax import lax
from jax.experimental import pallas as pl
from jax.experimental.pallas import tpu as pltpu
from jax.experimental.pallas import tpu_sc as plsc

N_NODES = 10000
D = 128
DH = D // 2
NC = 2
NS = 16
EB = 128
NBUF = 5
NHALF = 4
N_PAD = 10240
ROWS_PER_TILE = N_PAD // NS
RB = 1000


def _make_sc_agg(nb, with_counts):
  assert nb % (NHALF * NBUF) == 0
  nb2 = nb // NHALF
  nr2 = nb2 // NBUF
  mesh = plsc.VectorSubcoreMesh(core_axis_name="c", subcore_axis_name="s")

  out_type = [jax.ShapeDtypeStruct((NC, N_PAD, DH), jnp.float32)]
  scratch = [
      pltpu.VMEM((nb2, EB), jnp.int32),
      pltpu.VMEM((nb2, EB), jnp.int32),
  ]
  scratch += [pltpu.VMEM((EB, DH), jnp.float32) for _ in range(NBUF)]
  scratch += [pltpu.SemaphoreType.DMA for _ in range(3 * NBUF)]
  scratch += [
      pltpu.VMEM((EB, 16), jnp.float32),
      pltpu.VMEM((EB, 16), jnp.float32),
      pltpu.VMEM_SHARED((N_PAD, DH), jnp.float32),
  ]
  if with_counts:
    out_type.append(jax.ShapeDtypeStruct((NC, N_PAD, 16), jnp.float32))
    scratch.append(pltpu.VMEM_SHARED((N_PAD, 16), jnp.float32))

  def body(src_hbm, dst_hbm, table_hbm, zeros_hbm, zeros16_hbm, ones_hbm,
           *rest):
    if with_counts:
      p_hbm, cnt_hbm = rest[0], rest[1]
      rest = rest[2:]
    else:
      p_hbm = rest[0]
      rest = rest[1:]
    src_v, dst_v = rest[0], rest[1]
    rest = rest[2:]
    rows = list(rest[:NBUF])
    gsem = list(rest[NBUF:2 * NBUF])
    ssem = list(rest[2 * NBUF:3 * NBUF])
    csem = list(rest[3 * NBUF:4 * NBUF])
    rest = rest[4 * NBUF:]
    ones_v, cbuf, acc = rest[0], rest[1], rest[2]
    cacc = rest[3] if with_counts else None

    c = lax.axis_index("c")
    s = lax.axis_index("s")
    row0 = s * ROWS_PER_TILE
    table_c = table_hbm.at[c]

    def gather(b, j):
      pltpu.async_copy(table_c.at[src_v.at[j]], rows[b], gsem[b])

    def wait_gather(b):
      pltpu.make_async_copy(table_c.at[src_v.at[0]], rows[b],
                            gsem[b]).wait()

    def scatter(b, j):
      pltpu.async_copy(rows[b], acc.at[dst_v.at[j]], ssem[b], add=True)
      if with_counts:
        pltpu.async_copy(ones_v, cacc.at[dst_v.at[j]], csem[b], add=True)

    def wait_scatter(b):
      pltpu.make_async_copy(rows[b], acc.at[dst_v.at[0]], ssem[b]).wait()
      if with_counts:
        pltpu.make_async_copy(ones_v, cacc.at[dst_v.at[0]], csem[b]).wait()

    pltpu.sync_copy(zeros_hbm, rows[0])
    for t in range(ROWS_PER_TILE // 128):
      pltpu.sync_copy(rows[0], acc.at[pl.ds(row0 + t * 128, 128)])
    if with_counts:
      pltpu.sync_copy(ones_hbm, ones_v)
      pltpu.sync_copy(zeros16_hbm, cbuf)
      for t in range(ROWS_PER_TILE // 128):
        pltpu.sync_copy(cbuf, cacc.at[pl.ds(row0 + t * 128, 128)])

    for h in range(NHALF):
      pltpu.sync_copy(src_hbm.at[s, pl.ds(h * nb2, nb2)], src_v)
      pltpu.sync_copy(dst_hbm.at[s, pl.ds(h * nb2, nb2)], dst_v)

      for b in range(NBUF):
        gather(b, b)

      if h == 0:
        plsc.subcore_barrier()

      def round_body(r, carry):
        for b in range(NBUF):
          wait_gather(b)
          scatter(b, r * NBUF + b)
        for b in range(NBUF):
          wait_scatter(b)
          gather(b, jnp.minimum(r + 1, nr2 - 1) * NBUF + b)
        return carry

      lax.fori_loop(0, nr2, round_body, 0)
      for b in range(NBUF):
        wait_gather(b)

    plsc.subcore_barrier()

    for t in range(ROWS_PER_TILE // 128):
      sl = pl.ds(row0 + t * 128, 128)
      pltpu.sync_copy(acc.at[sl], rows[0])
      pltpu.sync_copy(rows[0], p_hbm.at[c, sl])
      if with_counts:
        pltpu.sync_copy(cacc.at[sl], cbuf)
        pltpu.sync_copy(cbuf, cnt_hbm.at[c, sl])

  return pl.kernel(
      body, out_type=tuple(out_type), mesh=mesh, scratch_types=scratch,
      compiler_params=pltpu.CompilerParams(use_tc_tiling_on_sc=False))


def _mm2_body(x_ref, w_ref, ol_ref, or_ref):
  y = jnp.dot(x_ref[...], w_ref[...], preferred_element_type=jnp.float32)
  ol_ref[0] = y[:, :DH]
  ol_ref[1] = y[:, DH:D]
  or_ref[...] = y[:, D:]


def _mid_body(p_ref, cnt_ref, xr_ref, b_ref, w_ref, ol_ref, or_ref):
  cnt = cnt_ref[0, :, 0:1]
  rc = 1.0 / jnp.maximum(cnt, 1.0)
  mean = jnp.concatenate([p_ref[0], p_ref[1]], axis=1) * rc
  h = jnp.maximum(mean + xr_ref[...] + b_ref[...], 0.0)
  y = jnp.dot(h, w_ref[...], preferred_element_type=jnp.float32)
  ol_ref[0] = y[:, :DH]
  ol_ref[1] = y[:, DH:D]
  or_ref[...] = y[:, D:]


def _fin_body(q_ref, cnt_ref, hr_ref, b_ref, o_ref):
  cnt = cnt_ref[0, :, 0:1]
  rc = 1.0 / jnp.maximum(cnt, 1.0)
  agg = jnp.concatenate([q_ref[0], q_ref[1]], axis=1)
  o_ref[...] = agg * rc + hr_ref[...] + b_ref[...]


def _row_spec(shape3=None):
  if shape3 is None:
    return pl.BlockSpec((RB, D), lambda i: (i, 0))
  return pl.BlockSpec(shape3, lambda i: (0, i, 0))


def _tc_mm2(x, wcat):
  grid = (N_NODES // RB,)
  return pl.pallas_call(
      _mm2_body,
      grid=grid,
      in_specs=[_row_spec(), pl.BlockSpec((D, 2 * D), lambda i: (0, 0))],
      out_specs=[_row_spec((NC, RB, DH)), _row_spec()],
      out_shape=[jax.ShapeDtypeStruct((NC, N_NODES, DH), jnp.float32),
                 jax.ShapeDtypeStruct((N_NODES, D), jnp.float32)],
  )(x, wcat)


def _tc_mid(p, cnt, xr, b, wcat):
  grid = (N_NODES // RB,)
  return pl.pallas_call(
      _mid_body,
      grid=grid,
      in_specs=[
          _row_spec((NC, RB, DH)),
          _row_spec((1, RB, 16)),
          _row_spec(),
          pl.BlockSpec((1, D), lambda i: (0, 0)),
          pl.BlockSpec((D, 2 * D), lambda i: (0, 0)),
      ],
      out_specs=[_row_spec((NC, RB, DH)), _row_spec()],
      out_shape=[jax.ShapeDtypeStruct((NC, N_NODES, DH), jnp.float32),
                 jax.ShapeDtypeStruct((N_NODES, D), jnp.float32)],
  )(p, cnt, xr, b, wcat)


def _tc_fin(q, cnt, hr, b):
  grid = (N_NODES // RB,)
  return pl.pallas_call(
      _fin_body,
      grid=grid,
      in_specs=[
          _row_spec((NC, RB, DH)),
          _row_spec((1, RB, 16)),
          _row_spec(),
          pl.BlockSpec((1, D), lambda i: (0, 0)),
      ],
      out_specs=_row_spec(),
      out_shape=jax.ShapeDtypeStruct((N_NODES, D), jnp.float32),
  )(q, cnt, hr, b)


@jax.jit
def _run(x, src, dst, W1l, b1, W1r, W2l, b2, W2r):
  e = src.shape[0]
  nb = -(-e // (NS * EB * NHALF * NBUF)) * NHALF * NBUF
  e_pad = NS * EB * nb
  src_p = jnp.concatenate(
      [src, jnp.zeros((e_pad - e,), jnp.int32)]).reshape(NS, nb, EB)
  dst_p = jnp.concatenate(
      [dst, jnp.full((e_pad - e,), N_NODES, jnp.int32)]).reshape(NS, nb, EB)

  zeros = jnp.zeros((128, DH), jnp.float32)
  zeros16 = jnp.zeros((128, 16), jnp.float32)
  ones16 = jnp.ones((128, 16), jnp.float32)
  wcat1 = jnp.concatenate([W1l.T, W1r.T], axis=1)
  wcat2 = jnp.concatenate([W2l.T, W2r.T], axis=1)

  agg_cnt = _make_sc_agg(nb, True)
  agg = _make_sc_agg(nb, False)

  xl, xr = _tc_mm2(x, wcat1)
  p, cnt = agg_cnt(src_p, dst_p, xl, zeros, zeros16, ones16)
  hl, hr = _tc_mid(p, cnt[:1], xr, b1.reshape(1, D), wcat2)
  (q,) = agg(src_p, dst_p, hl, zeros, zeros16, ones16)
  return _tc_fin(q, cnt[:1], hr, b2.reshape(1, D))


def kernel(x, edge_index, W1l, b1, W1r, W2l, b2, W2r):
  src = edge_index[0].astype(jnp.int32)
  dst = edge_index[1].astype(jnp.int32)
  return _run(x, src, dst, W1l, b1, W1r, W2l, b2, W2r)


# speedup vs baseline: 1.0209x; 1.0209x over previous
import functools

import jax
import jax.numpy as jnp
from jax import lax
from jax.experimental import pallas as pl
from jax.experimental.pallas import tpu as pltpu
from jax.experimental.pallas import tpu_sc as plsc

N_NODES = 10000
D = 128
DH = D // 2
CW = 16
NC = 2
NS = 16
EB = 128
NBUF = 4
N_PAD = 10240
ROWS_PER_TILE = N_PAD // NS
RB = 1000


def _make_sc_agg(nb, width):
  assert nb % (2 * NBUF) == 0
  nb2 = nb // 2
  mesh = plsc.VectorSubcoreMesh(core_axis_name="c", subcore_axis_name="s")

  out_type = jax.ShapeDtypeStruct((NC, N_PAD, width), jnp.float32)
  scratch = [
      pltpu.VMEM((nb2, EB), jnp.int32),
      pltpu.VMEM((nb2, EB), jnp.int32),
  ]
  scratch += [pltpu.VMEM((EB, width), jnp.float32) for _ in range(NBUF)]
  scratch += [pltpu.SemaphoreType.DMA for _ in range(NBUF)]
  scratch.append(pltpu.VMEM_SHARED((N_PAD, width), jnp.float32))

  def body(src_hbm, dst_hbm, table_hbm, zeros_hbm, p_hbm, src_v, dst_v,
           *rest):
    rows = list(rest[:NBUF])
    sems = list(rest[NBUF:2 * NBUF])
    acc = rest[2 * NBUF]

    c = lax.axis_index("c")
    s = lax.axis_index("s")
    row0 = s * ROWS_PER_TILE
    table_c = table_hbm.at[c]

    pltpu.sync_copy(zeros_hbm, rows[0])
    for t in range(ROWS_PER_TILE // 128):
      pltpu.sync_copy(rows[0], acc.at[pl.ds(row0 + t * 128, 128)])

    for h in range(2):
      pltpu.sync_copy(src_hbm.at[s, pl.ds(h * nb2, nb2)], src_v)
      pltpu.sync_copy(dst_hbm.at[s, pl.ds(h * nb2, nb2)], dst_v)

      for b in range(NBUF):
        pltpu.async_copy(table_c.at[src_v.at[b]], rows[b], sems[b])

      if h == 0:
        plsc.subcore_barrier()

      def loop_body(i, carry):
        for b in range(NBUF):
          j = NBUF * i + b
          pltpu.make_async_copy(table_c.at[src_v.at[b]], rows[b],
                                sems[b]).wait()
          pltpu.sync_copy(rows[b], acc.at[dst_v.at[j]], add=True)
          jn = jnp.minimum(j + NBUF, nb2 - 1)
          pltpu.async_copy(table_c.at[src_v.at[jn]], rows[b], sems[b])
        return carry

      lax.fori_loop(0, nb2 // NBUF, loop_body, 0)
      for b in range(NBUF):
        pltpu.make_async_copy(table_c.at[src_v.at[b]], rows[b],
                              sems[b]).wait()

    plsc.subcore_barrier()

    for t in range(ROWS_PER_TILE // 128):
      sl = pl.ds(row0 + t * 128, 128)
      pltpu.sync_copy(acc.at[sl], rows[0])
      pltpu.sync_copy(rows[0], p_hbm.at[c, sl])

  return pl.kernel(
      body, out_type=out_type, mesh=mesh, scratch_types=scratch,
      compiler_params=pltpu.CompilerParams(use_tc_tiling_on_sc=False))


def _mm2_body(x_ref, w_ref, ol_ref, or_ref):
  y = jnp.dot(x_ref[...], w_ref[...], preferred_element_type=jnp.float32)
  ones = jnp.ones((y.shape[0], CW), jnp.float32)
  ol_ref[0] = jnp.concatenate([y[:, :DH], ones], axis=1)
  ol_ref[1] = jnp.concatenate([y[:, DH:D], ones], axis=1)
  or_ref[...] = y[:, D:]


def _mid_body(p_ref, xr_ref, b_ref, w_ref, ol_ref, or_ref):
  cnt = p_ref[0, :, DH:DH + 1]
  rc = 1.0 / jnp.maximum(cnt, 1.0)
  mean = jnp.concatenate([p_ref[0, :, :DH], p_ref[1, :, :DH]], axis=1) * rc
  h = jnp.maximum(mean + xr_ref[...] + b_ref[...], 0.0)
  y = jnp.dot(h, w_ref[...], preferred_element_type=jnp.float32)
  ol_ref[0] = y[:, :DH]
  ol_ref[1] = y[:, DH:D]
  or_ref[...] = y[:, D:]


def _fin_body(q_ref, p_ref, hr_ref, b_ref, o_ref):
  cnt = p_ref[0, :, DH:DH + 1]
  rc = 1.0 / jnp.maximum(cnt, 1.0)
  agg = jnp.concatenate([q_ref[0], q_ref[1]], axis=1)
  o_ref[...] = agg * rc + hr_ref[...] + b_ref[...]


def _row_spec(shape3=None):
  if shape3 is None:
    return pl.BlockSpec((RB, D), lambda i: (i, 0))
  return pl.BlockSpec(shape3, lambda i: (0, i, 0))


def _tc_mm2(x, wcat):
  grid = (N_NODES // RB,)
  return pl.pallas_call(
      _mm2_body,
      grid=grid,
      in_specs=[_row_spec(), pl.BlockSpec((D, 2 * D), lambda i: (0, 0))],
      out_specs=[_row_spec((NC, RB, DH + CW)), _row_spec()],
      out_shape=[jax.ShapeDtypeStruct((NC, N_NODES, DH + CW), jnp.float32),
                 jax.ShapeDtypeStruct((N_NODES, D), jnp.float32)],
  )(x, wcat)


def _tc_mid(p, xr, b, wcat):
  grid = (N_NODES // RB,)
  return pl.pallas_call(
      _mid_body,
      grid=grid,
      in_specs=[
          _row_spec((NC, RB, DH + CW)),
          _row_spec(),
          pl.BlockSpec((1, D), lambda i: (0, 0)),
          pl.BlockSpec((D, 2 * D), lambda i: (0, 0)),
      ],
      out_specs=[_row_spec((NC, RB, DH)), _row_spec()],
      out_shape=[jax.ShapeDtypeStruct((NC, N_NODES, DH), jnp.float32),
                 jax.ShapeDtypeStruct((N_NODES, D), jnp.float32)],
  )(p, xr, b, wcat)


def _tc_fin(q, p, hr, b):
  grid = (N_NODES // RB,)
  return pl.pallas_call(
      _fin_body,
      grid=grid,
      in_specs=[
          _row_spec((NC, RB, DH)),
          _row_spec((1, RB, DH + CW)),
          _row_spec(),
          pl.BlockSpec((1, D), lambda i: (0, 0)),
      ],
      out_specs=_row_spec(),
      out_shape=jax.ShapeDtypeStruct((N_NODES, D), jnp.float32),
  )(q, p, hr, b)


@jax.jit
def _run(x, src, dst, W1l, b1, W1r, W2l, b2, W2r):
  e = src.shape[0]
  nb = -(-e // (NS * EB * 2 * NBUF)) * 2 * NBUF
  e_pad = NS * EB * nb
  src_p = jnp.concatenate(
      [src, jnp.zeros((e_pad - e,), jnp.int32)]).reshape(NS, nb, EB)
  dst_p = jnp.concatenate(
      [dst, jnp.full((e_pad - e,), N_NODES, jnp.int32)]).reshape(NS, nb, EB)

  zeros80 = jnp.zeros((128, DH + CW), jnp.float32)
  zeros64 = jnp.zeros((128, DH), jnp.float32)
  wcat1 = jnp.concatenate([W1l.T, W1r.T], axis=1)
  wcat2 = jnp.concatenate([W2l.T, W2r.T], axis=1)

  agg1 = _make_sc_agg(nb, DH + CW)
  agg2 = _make_sc_agg(nb, DH)

  xl, xr = _tc_mm2(x, wcat1)
  p = agg1(src_p, dst_p, xl, zeros80)
  hl, hr = _tc_mid(p, xr, b1.reshape(1, D), wcat2)
  q = agg2(src_p, dst_p, hl, zeros64)
  return _tc_fin(q, p[:1], hr, b2.reshape(1, D))


def kernel(x, edge_index, W1l, b1, W1r, W2l, b2, W2r):
  src = edge_index[0].astype(jnp.int32)
  dst = edge_index[1].astype(jnp.int32)
  return _run(x, src, dst, W1l, b1, W1r, W2l, b2, W2r)


# speedup vs baseline: 1.0917x; 1.0693x over previous
import functools

import jax
import jax.numpy as jnp
from jax import lax
from jax.experimental import pallas as pl
from jax.experimental.pallas import tpu as pltpu
from jax.experimental.pallas import tpu_sc as plsc

N_NODES = 10000
D = 128
DH = D // 2
NC = 2
NS = 16
EB = 128
NBUF = 4
N_PAD = 10240
ROWS_PER_TILE = N_PAD // NS
RB = 1000


def _make_sc_agg(nb, with_counts):
  assert nb % NBUF == 0
  mesh = plsc.VectorSubcoreMesh(core_axis_name="c", subcore_axis_name="s")

  out_type = [jax.ShapeDtypeStruct((NC, N_PAD, DH), jnp.float32)]
  scratch = [
      pltpu.VMEM((nb, EB), jnp.int32),
      pltpu.VMEM((nb, EB), jnp.int32),
  ]
  scratch += [pltpu.VMEM((EB, DH), jnp.float32) for _ in range(NBUF)]
  scratch += [pltpu.SemaphoreType.DMA for _ in range(NBUF)]
  scratch += [
      pltpu.VMEM((EB, 16), jnp.float32),
      pltpu.VMEM((EB, 16), jnp.float32),
      pltpu.VMEM_SHARED((N_PAD, DH), jnp.float32),
  ]
  if with_counts:
    out_type.append(jax.ShapeDtypeStruct((NC, N_PAD, 16), jnp.float32))
    scratch.append(pltpu.VMEM_SHARED((N_PAD, 16), jnp.float32))

  def body(src_hbm, dst_hbm, table_hbm, zeros_hbm, zeros16_hbm, ones_hbm,
           *rest):
    if with_counts:
      p_hbm, cnt_hbm = rest[0], rest[1]
      rest = rest[2:]
    else:
      p_hbm = rest[0]
      rest = rest[1:]
    src_v, dst_v = rest[0], rest[1]
    rows = list(rest[2:2 + NBUF])
    sems = list(rest[2 + NBUF:2 + 2 * NBUF])
    ones_v = rest[2 + 2 * NBUF]
    cbuf = rest[3 + 2 * NBUF]
    acc = rest[4 + 2 * NBUF]
    cacc = rest[5 + 2 * NBUF] if with_counts else None

    c = lax.axis_index("c")
    s = lax.axis_index("s")
    row0 = s * ROWS_PER_TILE
    table_c = table_hbm.at[c]

    pltpu.sync_copy(src_hbm.at[s], src_v)
    pltpu.sync_copy(dst_hbm.at[s], dst_v)

    pltpu.sync_copy(zeros_hbm, rows[0])
    for t in range(ROWS_PER_TILE // 128):
      pltpu.sync_copy(rows[0], acc.at[pl.ds(row0 + t * 128, 128)])
    if with_counts:
      pltpu.sync_copy(ones_hbm, ones_v)
      pltpu.sync_copy(zeros16_hbm, cbuf)
      for t in range(ROWS_PER_TILE // 128):
        pltpu.sync_copy(cbuf, cacc.at[pl.ds(row0 + t * 128, 128)])

    for b in range(NBUF):
      pltpu.async_copy(table_c.at[src_v.at[b]], rows[b], sems[b])

    plsc.subcore_barrier()

    def loop_body(i, carry):
      for b in range(NBUF):
        j = NBUF * i + b
        pltpu.make_async_copy(table_c.at[src_v.at[b]], rows[b],
                              sems[b]).wait()
        pltpu.sync_copy(rows[b], acc.at[dst_v.at[j]], add=True)
        if with_counts:
          pltpu.sync_copy(ones_v, cacc.at[dst_v.at[j]], add=True)
        jn = jnp.minimum(j + NBUF, nb - 1)
        pltpu.async_copy(table_c.at[src_v.at[jn]], rows[b], sems[b])
      return carry

    lax.fori_loop(0, nb // NBUF, loop_body, 0)
    for b in range(NBUF):
      pltpu.make_async_copy(table_c.at[src_v.at[b]], rows[b],
                            sems[b]).wait()

    plsc.subcore_barrier()

    for t in range(ROWS_PER_TILE // 128):
      sl = pl.ds(row0 + t * 128, 128)
      pltpu.sync_copy(acc.at[sl], rows[0])
      pltpu.sync_copy(rows[0], p_hbm.at[c, sl])
      if with_counts:
        pltpu.sync_copy(cacc.at[sl], cbuf)
        pltpu.sync_copy(cbuf, cnt_hbm.at[c, sl])

  return pl.kernel(
      body, out_type=tuple(out_type), mesh=mesh, scratch_types=scratch,
      compiler_params=pltpu.CompilerParams(use_tc_tiling_on_sc=False))


def _mm2_body(x_ref, w_ref, ol_ref, or_ref):
  y = jnp.dot(x_ref[...], w_ref[...], preferred_element_type=jnp.float32)
  ol_ref[0] = y[:, :DH]
  ol_ref[1] = y[:, DH:D]
  or_ref[...] = y[:, D:]


def _mid_body(p_ref, cnt_ref, xr_ref, b_ref, w_ref, ol_ref, or_ref):
  cnt = cnt_ref[0, :, 0:1]
  rc = 1.0 / jnp.maximum(cnt, 1.0)
  mean = jnp.concatenate([p_ref[0], p_ref[1]], axis=1) * rc
  h = jnp.maximum(mean + xr_ref[...] + b_ref[...], 0.0)
  y = jnp.dot(h, w_ref[...], preferred_element_type=jnp.float32)
  ol_ref[0] = y[:, :DH]
  ol_ref[1] = y[:, DH:D]
  or_ref[...] = y[:, D:]


def _fin_body(q_ref, cnt_ref, hr_ref, b_ref, o_ref):
  cnt = cnt_ref[0, :, 0:1]
  rc = 1.0 / jnp.maximum(cnt, 1.0)
  agg = jnp.concatenate([q_ref[0], q_ref[1]], axis=1)
  o_ref[...] = agg * rc + hr_ref[...] + b_ref[...]


def _row_spec(shape3=None):
  if shape3 is None:
    return pl.BlockSpec((RB, D), lambda i: (i, 0))
  return pl.BlockSpec(shape3, lambda i: (0, i, 0))


def _tc_mm2(x, wcat):
  grid = (N_NODES // RB,)
  return pl.pallas_call(
      _mm2_body,
      grid=grid,
      in_specs=[_row_spec(), pl.BlockSpec((D, 2 * D), lambda i: (0, 0))],
      out_specs=[_row_spec((NC, RB, DH)), _row_spec()],
      out_shape=[jax.ShapeDtypeStruct((NC, N_NODES, DH), jnp.float32),
                 jax.ShapeDtypeStruct((N_NODES, D), jnp.float32)],
  )(x, wcat)


def _tc_mid(p, cnt, xr, b, wcat):
  grid = (N_NODES // RB,)
  return pl.pallas_call(
      _mid_body,
      grid=grid,
      in_specs=[
          _row_spec((NC, RB, DH)),
          _row_spec((1, RB, 16)),
          _row_spec(),
          pl.BlockSpec((1, D), lambda i: (0, 0)),
          pl.BlockSpec((D, 2 * D), lambda i: (0, 0)),
      ],
      out_specs=[_row_spec((NC, RB, DH)), _row_spec()],
      out_shape=[jax.ShapeDtypeStruct((NC, N_NODES, DH), jnp.float32),
                 jax.ShapeDtypeStruct((N_NODES, D), jnp.float32)],
  )(p, cnt, xr, b, wcat)


def _tc_fin(q, cnt, hr, b):
  grid = (N_NODES // RB,)
  return pl.pallas_call(
      _fin_body,
      grid=grid,
      in_specs=[
          _row_spec((NC, RB, DH)),
          _row_spec((1, RB, 16)),
          _row_spec(),
          pl.BlockSpec((1, D), lambda i: (0, 0)),
      ],
      out_specs=_row_spec(),
      out_shape=jax.ShapeDtypeStruct((N_NODES, D), jnp.float32),
  )(q, cnt, hr, b)


@jax.jit
def _run(x, src, dst, W1l, b1, W1r, W2l, b2, W2r):
  e = src.shape[0]
  nb = -(-e // (NS * EB * NBUF)) * NBUF
  e_pad = NS * EB * nb
  src_p = jnp.concatenate(
      [src, jnp.zeros((e_pad - e,), jnp.int32)]).reshape(NS, nb, EB)
  dst_p = jnp.concatenate(
      [dst, jnp.full((e_pad - e,), N_NODES, jnp.int32)]).reshape(NS, nb, EB)

  zeros = jnp.zeros((128, DH), jnp.float32)
  zeros16 = jnp.zeros((128, 16), jnp.float32)
  ones16 = jnp.ones((128, 16), jnp.float32)
  wcat1 = jnp.concatenate([W1l.T, W1r.T], axis=1)
  wcat2 = jnp.concatenate([W2l.T, W2r.T], axis=1)

  agg_cnt = _make_sc_agg(nb, True)
  agg = _make_sc_agg(nb, False)

  xl, xr = _tc_mm2(x, wcat1)
  p, cnt = agg_cnt(src_p, dst_p, xl, zeros, zeros16, ones16)
  hl, hr = _tc_mid(p, cnt, xr, b1.reshape(1, D), wcat2)
  (q,) = agg(src_p, dst_p, hl, zeros, zeros16, ones16)
  return _tc_fin(q, cnt, hr, b2.reshape(1, D))


def kernel(x, edge_index, W1l, b1, W1r, W2l, b2, W2r):
  src = edge_index[0].astype(jnp.int32)
  dst = edge_index[1].astype(jnp.int32)
  return _run(x, src, dst, W1l, b1, W1r, W2l, b2, W2r)


# speedup vs baseline: 1.0998x; 1.0074x over previous
import functools

import jax
import jax.numpy as jnp
from jax import lax
from jax.experimental import pallas as pl
from jax.experimental.pallas import tpu as pltpu
from jax.experimental.pallas import tpu_sc as plsc

N_NODES = 10000
D = 128
DH = D // 2
NC = 2
NS = 16
EB = 128
NBUF = 4
N_PAD = 10240
ROWS_PER_TILE = N_PAD // NS
RB = 1000


def _make_sc_agg(nb, with_counts):
  assert nb % (2 * NBUF) == 0
  nb2 = nb // 2
  mesh = plsc.VectorSubcoreMesh(core_axis_name="c", subcore_axis_name="s")

  out_type = [jax.ShapeDtypeStruct((NC, N_PAD, DH), jnp.float32)]
  scratch = [
      pltpu.VMEM((nb2, EB), jnp.int32),
      pltpu.VMEM((nb2, EB), jnp.int32),
  ]
  scratch += [pltpu.VMEM((EB, DH), jnp.float32) for _ in range(NBUF)]
  scratch += [pltpu.SemaphoreType.DMA for _ in range(NBUF)]
  scratch += [
      pltpu.VMEM((EB, 16), jnp.float32),
      pltpu.VMEM((EB, 16), jnp.float32),
      pltpu.VMEM_SHARED((N_PAD, DH), jnp.float32),
  ]
  if with_counts:
    out_type.append(jax.ShapeDtypeStruct((NC, N_PAD, 16), jnp.float32))
    scratch.append(pltpu.VMEM_SHARED((N_PAD, 16), jnp.float32))

  def body(src_hbm, dst_hbm, table_hbm, zeros_hbm, zeros16_hbm, ones_hbm,
           *rest):
    if with_counts:
      p_hbm, cnt_hbm = rest[0], rest[1]
      rest = rest[2:]
    else:
      p_hbm = rest[0]
      rest = rest[1:]
    src_v, dst_v = rest[0], rest[1]
    rows = list(rest[2:2 + NBUF])
    sems = list(rest[2 + NBUF:2 + 2 * NBUF])
    ones_v = rest[2 + 2 * NBUF]
    cbuf = rest[3 + 2 * NBUF]
    acc = rest[4 + 2 * NBUF]
    cacc = rest[5 + 2 * NBUF] if with_counts else None

    c = lax.axis_index("c")
    s = lax.axis_index("s")
    row0 = s * ROWS_PER_TILE
    table_c = table_hbm.at[c]

    pltpu.sync_copy(zeros_hbm, rows[0])
    for t in range(ROWS_PER_TILE // 128):
      pltpu.sync_copy(rows[0], acc.at[pl.ds(row0 + t * 128, 128)])
    if with_counts:
      pltpu.sync_copy(ones_hbm, ones_v)
      pltpu.sync_copy(zeros16_hbm, cbuf)
      for t in range(ROWS_PER_TILE // 128):
        pltpu.sync_copy(cbuf, cacc.at[pl.ds(row0 + t * 128, 128)])

    for h in range(2):
      pltpu.sync_copy(src_hbm.at[s, pl.ds(h * nb2, nb2)], src_v)
      pltpu.sync_copy(dst_hbm.at[s, pl.ds(h * nb2, nb2)], dst_v)

      for b in range(NBUF):
        pltpu.async_copy(table_c.at[src_v.at[b]], rows[b], sems[b])

      if h == 0:
        plsc.subcore_barrier()

      def loop_body(i, carry):
        for b in range(NBUF):
          j = NBUF * i + b
          pltpu.make_async_copy(table_c.at[src_v.at[b]], rows[b],
                                sems[b]).wait()
          pltpu.sync_copy(rows[b], acc.at[dst_v.at[j]], add=True)
          if with_counts:
            pltpu.sync_copy(ones_v, cacc.at[dst_v.at[j]], add=True)
          jn = jnp.minimum(j + NBUF, nb2 - 1)
          pltpu.async_copy(table_c.at[src_v.at[jn]], rows[b], sems[b])
        return carry

      lax.fori_loop(0, nb2 // NBUF, loop_body, 0)
      for b in range(NBUF):
        pltpu.make_async_copy(table_c.at[src_v.at[b]], rows[b],
                              sems[b]).wait()

    plsc.subcore_barrier()

    for t in range(ROWS_PER_TILE // 128):
      sl = pl.ds(row0 + t * 128, 128)
      pltpu.sync_copy(acc.at[sl], rows[0])
      pltpu.sync_copy(rows[0], p_hbm.at[c, sl])
      if with_counts:
        pltpu.sync_copy(cacc.at[sl], cbuf)
        pltpu.sync_copy(cbuf, cnt_hbm.at[c, sl])

  return pl.kernel(
      body, out_type=tuple(out_type), mesh=mesh, scratch_types=scratch,
      compiler_params=pltpu.CompilerParams(use_tc_tiling_on_sc=False))


def _mm2_body(x_ref, w_ref, ol_ref, or_ref):
  y = jnp.dot(x_ref[...], w_ref[...], preferred_element_type=jnp.float32)
  ol_ref[0] = y[:, :DH]
  ol_ref[1] = y[:, DH:D]
  or_ref[...] = y[:, D:]


def _mid_body(p_ref, cnt_ref, xr_ref, b_ref, w_ref, ol_ref, or_ref):
  cnt = cnt_ref[0, :, 0:1]
  rc = 1.0 / jnp.maximum(cnt, 1.0)
  mean = jnp.concatenate([p_ref[0], p_ref[1]], axis=1) * rc
  h = jnp.maximum(mean + xr_ref[...] + b_ref[...], 0.0)
  y = jnp.dot(h, w_ref[...], preferred_element_type=jnp.float32)
  ol_ref[0] = y[:, :DH]
  ol_ref[1] = y[:, DH:D]
  or_ref[...] = y[:, D:]


def _fin_body(q_ref, cnt_ref, hr_ref, b_ref, o_ref):
  cnt = cnt_ref[0, :, 0:1]
  rc = 1.0 / jnp.maximum(cnt, 1.0)
  agg = jnp.concatenate([q_ref[0], q_ref[1]], axis=1)
  o_ref[...] = agg * rc + hr_ref[...] + b_ref[...]


def _row_spec(shape3=None):
  if shape3 is None:
    return pl.BlockSpec((RB, D), lambda i: (i, 0))
  return pl.BlockSpec(shape3, lambda i: (0, i, 0))


def _tc_mm2(x, wcat):
  grid = (N_NODES // RB,)
  return pl.pallas_call(
      _mm2_body,
      grid=grid,
      in_specs=[_row_spec(), pl.BlockSpec((D, 2 * D), lambda i: (0, 0))],
      out_specs=[_row_spec((NC, RB, DH)), _row_spec()],
      out_shape=[jax.ShapeDtypeStruct((NC, N_NODES, DH), jnp.float32),
                 jax.ShapeDtypeStruct((N_NODES, D), jnp.float32)],
  )(x, wcat)


def _tc_mid(p, cnt, xr, b, wcat):
  grid = (N_NODES // RB,)
  return pl.pallas_call(
      _mid_body,
      grid=grid,
      in_specs=[
          _row_spec((NC, RB, DH)),
          _row_spec((1, RB, 16)),
          _row_spec(),
          pl.BlockSpec((1, D), lambda i: (0, 0)),
          pl.BlockSpec((D, 2 * D), lambda i: (0, 0)),
      ],
      out_specs=[_row_spec((NC, RB, DH)), _row_spec()],
      out_shape=[jax.ShapeDtypeStruct((NC, N_NODES, DH), jnp.float32),
                 jax.ShapeDtypeStruct((N_NODES, D), jnp.float32)],
  )(p, cnt, xr, b, wcat)


def _tc_fin(q, cnt, hr, b):
  grid = (N_NODES // RB,)
  return pl.pallas_call(
      _fin_body,
      grid=grid,
      in_specs=[
          _row_spec((NC, RB, DH)),
          _row_spec((1, RB, 16)),
          _row_spec(),
          pl.BlockSpec((1, D), lambda i: (0, 0)),
      ],
      out_specs=_row_spec(),
      out_shape=jax.ShapeDtypeStruct((N_NODES, D), jnp.float32),
  )(q, cnt, hr, b)


@jax.jit
def _run(x, src, dst, W1l, b1, W1r, W2l, b2, W2r):
  e = src.shape[0]
  nb = -(-e // (NS * EB * 2 * NBUF)) * 2 * NBUF
  e_pad = NS * EB * nb
  src_p = jnp.concatenate(
      [src, jnp.zeros((e_pad - e,), jnp.int32)]).reshape(NS, nb, EB)
  dst_p = jnp.concatenate(
      [dst, jnp.full((e_pad - e,), N_NODES, jnp.int32)]).reshape(NS, nb, EB)

  zeros = jnp.zeros((128, DH), jnp.float32)
  zeros16 = jnp.zeros((128, 16), jnp.float32)
  ones16 = jnp.ones((128, 16), jnp.float32)
  wcat1 = jnp.concatenate([W1l.T, W1r.T], axis=1)
  wcat2 = jnp.concatenate([W2l.T, W2r.T], axis=1)

  agg_cnt = _make_sc_agg(nb, True)
  agg = _make_sc_agg(nb, False)

  xl, xr = _tc_mm2(x, wcat1)
  p, cnt = agg_cnt(src_p, dst_p, xl, zeros, zeros16, ones16)
  hl, hr = _tc_mid(p, cnt, xr, b1.reshape(1, D), wcat2)
  (q,) = agg(src_p, dst_p, hl, zeros, zeros16, ones16)
  return _tc_fin(q, cnt, hr, b2.reshape(1, D))


def kernel(x, edge_index, W1l, b1, W1r, W2l, b2, W2r):
  src = edge_index[0].astype(jnp.int32)
  dst = edge_index[1].astype(jnp.int32)
  return _run(x, src, dst, W1l, b1, W1r, W2l, b2, W2r)
